# baseline (speedup 1.0000x reference)
import functools

import jax
import jax.numpy as jnp
from jax import lax
from jax.experimental import pallas as pl
from jax.experimental.pallas import tpu as pltpu
from jax.experimental.pallas import tpu_sc as plsc

N_ITEMS = 8192
BATCH_N = 1024
H1_PAD = 640
LAT = 200
LAT_PAD = 256
TAU_C = 0.2

_NC = 2
_NS = 16
_NW = _NC * _NS
_SPLIT = 2
_D2 = N_ITEMS // _SPLIT
_B2 = BATCH_N * _SPLIT
_BPW = _B2 // _NW
_CH = 8
_NCH = _BPW // _CH
_NBUF = 3


@functools.cache
def _make_sc_gather():
    @functools.partial(
        pl.kernel,
        mesh=plsc.VectorSubcoreMesh(core_axis_name="c", subcore_axis_name="s"),
        out_type=jax.ShapeDtypeStruct((_B2, _D2), jnp.float32),
        scratch_types=[
            pltpu.VMEM((_BPW,), jnp.int32),
        ] + [pltpu.VMEM((_CH, _D2), jnp.float32) for _ in range(_NBUF)]
          + [pltpu.SemaphoreType.DMA for _ in range(_NBUF)],
    )
    def _sc_gather(gram_hbm, idx_hbm, out_hbm, idx_v, *bufsems):
        bufs, sems = bufsems[:_NBUF], bufsems[_NBUF:]
        wid = lax.axis_index("s") * _NC + lax.axis_index("c")
        base = wid * _BPW
        pltpu.sync_copy(idx_hbm.at[pl.ds(base, _BPW)], idx_v)
        copies = [
            pltpu.async_copy(
                gram_hbm.at[idx_v.at[pl.ds(c * _CH, _CH)]], bufs[c], sems[c])
            for c in range(_NBUF)]
        for c in range(_NCH):
            copies[c].wait()
            pltpu.sync_copy(bufs[c % _NBUF],
                            out_hbm.at[pl.ds(base + c * _CH, _CH)])
            if c + _NBUF < _NCH:
                copies.append(pltpu.async_copy(
                    gram_hbm.at[idx_v.at[pl.ds((c + _NBUF) * _CH, _CH)]],
                    bufs[(c + _NBUF) % _NBUF], sems[(c + _NBUF) % _NBUF]))

    return _sc_gather


_MBLK = 64
_GRID = BATCH_N // _MBLK


def _tc_body(gath_ref, rate_ref, w1_ref, b1_ref, w2m_ref, w2l_ref, b2m_ref,
             b2l_ref, itT_ref, bi0_ref, z_ref, out_ref, kl_ref, csc_ref):
    i = pl.program_id(0)

    @pl.when(i == 0)
    def _init():
        itT = itT_ref[...]
        cn = jnp.sum(itT * itT, axis=0, keepdims=True)
        csc_ref[...] = 1.0 / jnp.maximum(jnp.sqrt(cn), 1e-12)
        kl_ref[...] = jnp.zeros((1, 1), jnp.float32)

    x = gath_ref[...] * (rate_ref[...] > 0).astype(jnp.float32)
    ss = jnp.sum(x * x, axis=1, keepdims=True)
    bi0 = x * (1.0 / jnp.maximum(jnp.sqrt(ss), 1e-12))
    bi0_ref[...] = bi0
    h = jnp.tanh(jnp.dot(bi0, w1_ref[...], preferred_element_type=jnp.float32)
                 + b1_ref[...])
    mean = (jnp.dot(h, w2m_ref[...], preferred_element_type=jnp.float32)
            + b2m_ref[...])
    logvar = (jnp.dot(h, w2l_ref[...], preferred_element_type=jnp.float32)
              + b2l_ref[...])
    z_ref[...] = mean
    kl_ref[...] += jnp.sum(
        mean * mean + jnp.exp(logvar) - 1.0 - logvar).reshape(1, 1)
    zs = jnp.sum(mean * mean, axis=1, keepdims=True)
    zn = mean * (1.0 / jnp.maximum(jnp.sqrt(zs), 1e-12))
    out_ref[...] = (jnp.dot(zn, itT_ref[...], preferred_element_type=jnp.float32)
                    * csc_ref[...]) * (1.0 / TAU_C)


def _tc_encoder(gathered, rating, W1p, b1p, W2m, W2l, b2m, b2l, itT):
    full = lambda shp: pl.BlockSpec(shp, lambda i: (0, 0))
    blk = lambda shp: pl.BlockSpec(shp, lambda i: (i, 0))
    return pl.pallas_call(
        _tc_body,
        grid=(_GRID,),
        in_specs=[
            blk((_MBLK, N_ITEMS)),
            blk((_MBLK, N_ITEMS)),
            full((N_ITEMS, H1_PAD)),
            full((1, H1_PAD)),
            full((H1_PAD, LAT_PAD)),
            full((H1_PAD, LAT_PAD)),
            full((1, LAT_PAD)),
            full((1, LAT_PAD)),
            full((LAT_PAD, N_ITEMS)),
        ],
        out_specs=[
            blk((_MBLK, N_ITEMS)),
            blk((_MBLK, LAT_PAD)),
            blk((_MBLK, N_ITEMS)),
            pl.BlockSpec((1, 1), lambda i: (0, 0)),
        ],
        out_shape=[
            jax.ShapeDtypeStruct((BATCH_N, N_ITEMS), jnp.float32),
            jax.ShapeDtypeStruct((BATCH_N, LAT_PAD), jnp.float32),
            jax.ShapeDtypeStruct((BATCH_N, N_ITEMS), jnp.float32),
            jax.ShapeDtypeStruct((1, 1), jnp.float32),
        ],
        scratch_shapes=[pltpu.VMEM((1, N_ITEMS), jnp.float32)],
    )(gathered, rating, W1p, b1p, W2m, W2l, b2m, b2l, itT)


def kernel(rating_matrix_batch, rating_matrix_batch2, gram_matrix, W1, b1,
           W2, b2, items):
    idx = rating_matrix_batch2.astype(jnp.int32)
    idx2 = (idx[:, None] * _SPLIT
            + jnp.arange(_SPLIT, dtype=jnp.int32)[None, :]).reshape(_B2)
    gram2 = gram_matrix.reshape(N_ITEMS * _SPLIT, _D2)
    gathered2 = _make_sc_gather()(gram2, idx2)
    gathered = gathered2.reshape(BATCH_N, N_ITEMS)

    W1p = jnp.pad(W1, ((0, 0), (0, H1_PAD - W1.shape[1])))
    b1p = jnp.pad(b1, (0, H1_PAD - b1.shape[0])).reshape(1, H1_PAD)
    W2m = jnp.pad(W2[:, :LAT], ((0, H1_PAD - W2.shape[0]), (0, LAT_PAD - LAT)))
    W2l = jnp.pad(W2[:, LAT:], ((0, H1_PAD - W2.shape[0]), (0, LAT_PAD - LAT)))
    b2m = jnp.pad(b2[:LAT], (0, LAT_PAD - LAT)).reshape(1, LAT_PAD)
    b2l = jnp.pad(b2[LAT:], (0, LAT_PAD - LAT)).reshape(1, LAT_PAD)
    itT = jnp.pad(items.T, ((0, LAT_PAD - LAT), (0, 0)))

    bi0, z_p, out, klp = _tc_encoder(
        gathered, rating_matrix_batch, W1p, b1p, W2m, W2l, b2m, b2l, itT)
    z = z_p[:, :LAT]
    kl = 0.5 * klp[0, 0] / BATCH_N
    return (z, out, kl, bi0)


# speedup vs baseline: 3.0889x; 3.0889x over previous
import functools

import jax
import jax.numpy as jnp
from jax import lax
from jax.experimental import pallas as pl
from jax.experimental.pallas import tpu as pltpu
from jax.experimental.pallas import tpu_sc as plsc

N_ITEMS = 8192
BATCH_N = 1024
H1_PAD = 640
LAT = 200
LAT_PAD = 256
TAU_C = 0.2

_NC = 2
_NS = 16
_NW = _NC * _NS
_BPW = BATCH_N // _NW
_CH = 8
_HALF = N_ITEMS // 2
_NCH = _BPW // _CH
_NU = _NCH * 2
_NBUF = 3


@functools.cache
def _make_sc_gather():
    @functools.partial(
        pl.kernel,
        mesh=plsc.VectorSubcoreMesh(core_axis_name="c", subcore_axis_name="s"),
        out_type=jax.ShapeDtypeStruct((BATCH_N, N_ITEMS), jnp.float32),
        scratch_types=[
            pltpu.VMEM((_BPW,), jnp.int32),
        ] + [pltpu.VMEM((_CH, _HALF), jnp.float32) for _ in range(_NBUF)]
          + [pltpu.SemaphoreType.DMA for _ in range(_NBUF)],
    )
    def _sc_gather(gram_hbm, idx_hbm, out_hbm, idx_v, *bufsems):
        bufs, sems = bufsems[:_NBUF], bufsems[_NBUF:]
        wid = lax.axis_index("s") * _NC + lax.axis_index("c")
        base = wid * _BPW

        def unit_src(u):
            c, h = u // 2, u % 2
            return gram_hbm.at[idx_v.at[pl.ds(c * _CH, _CH)],
                               pl.ds(h * _HALF, _HALF)]

        def unit_dst(u):
            c, h = u // 2, u % 2
            return out_hbm.at[pl.ds(base + c * _CH, _CH),
                              pl.ds(h * _HALF, _HALF)]

        pltpu.sync_copy(idx_hbm.at[pl.ds(base, _BPW)], idx_v)
        copies = [pltpu.async_copy(unit_src(u), bufs[u], sems[u])
                  for u in range(_NBUF)]
        for u in range(_NU):
            copies[u].wait()
            pltpu.sync_copy(bufs[u % _NBUF], unit_dst(u))
            if u + _NBUF < _NU:
                copies.append(pltpu.async_copy(
                    unit_src(u + _NBUF),
                    bufs[(u + _NBUF) % _NBUF], sems[(u + _NBUF) % _NBUF]))

    return _sc_gather


_MBLK = 64
_GRID = BATCH_N // _MBLK


def _dot_t(a, b):
    return lax.dot_general(a, b, (((1,), (1,)), ((), ())),
                           preferred_element_type=jnp.float32)


def _tc_body(gath_ref, rate_ref, w1_ref, b1_ref, w2m_ref, w2l_ref, b2m_ref,
             b2l_ref, items_ref, bi0_ref, z_ref, out_ref, kl_ref, csc_ref):
    i = pl.program_id(0)

    @pl.when(i == 0)
    def _init():
        it = items_ref[...]
        cn = _dot_t(jnp.ones((8, LAT), jnp.float32), it * it)[:1]
        csc_ref[...] = 1.0 / jnp.maximum(jnp.sqrt(cn), 1e-12)
        kl_ref[...] = jnp.zeros((1, 1), jnp.float32)

    x = gath_ref[...] * (rate_ref[...] > 0).astype(jnp.float32)
    ss = jnp.sum(x * x, axis=1, keepdims=True)
    bi0 = x * (1.0 / jnp.maximum(jnp.sqrt(ss), 1e-12))
    bi0_ref[...] = bi0
    h = jnp.tanh(jnp.dot(bi0, w1_ref[...], preferred_element_type=jnp.float32)
                 + b1_ref[...])
    mean = (jnp.dot(h, w2m_ref[...], preferred_element_type=jnp.float32)
            + b2m_ref[...])
    logvar = (jnp.dot(h, w2l_ref[...], preferred_element_type=jnp.float32)
              + b2l_ref[...])
    z_ref[...] = mean
    kl_ref[...] += jnp.sum(
        mean * mean + jnp.exp(logvar) - 1.0 - logvar).reshape(1, 1)
    zs = jnp.sum(mean * mean, axis=1, keepdims=True)
    zn = mean * (1.0 / jnp.maximum(jnp.sqrt(zs), 1e-12))
    out_ref[...] = (_dot_t(zn[:, :LAT], items_ref[...])
                    * csc_ref[...]) * (1.0 / TAU_C)


def _tc_encoder(gathered, rating, W1, b1r, W2m, W2l, b2m, b2l, items):
    full = lambda shp: pl.BlockSpec(shp, lambda i: (0, 0))
    blk = lambda shp: pl.BlockSpec(shp, lambda i: (i, 0))
    return pl.pallas_call(
        _tc_body,
        grid=(_GRID,),
        in_specs=[
            blk((_MBLK, N_ITEMS)),
            blk((_MBLK, N_ITEMS)),
            full((N_ITEMS, 600)),
            full((1, 600)),
            full((600, LAT_PAD)),
            full((600, LAT_PAD)),
            full((1, LAT_PAD)),
            full((1, LAT_PAD)),
            full((N_ITEMS, LAT)),
        ],
        out_specs=[
            blk((_MBLK, N_ITEMS)),
            blk((_MBLK, LAT_PAD)),
            blk((_MBLK, N_ITEMS)),
            pl.BlockSpec((1, 1), lambda i: (0, 0)),
        ],
        out_shape=[
            jax.ShapeDtypeStruct((BATCH_N, N_ITEMS), jnp.float32),
            jax.ShapeDtypeStruct((BATCH_N, LAT_PAD), jnp.float32),
            jax.ShapeDtypeStruct((BATCH_N, N_ITEMS), jnp.float32),
            jax.ShapeDtypeStruct((1, 1), jnp.float32),
        ],
        scratch_shapes=[pltpu.VMEM((1, N_ITEMS), jnp.float32)],
    )(gathered, rating, W1, b1r, W2m, W2l, b2m, b2l, items)


def kernel(rating_matrix_batch, rating_matrix_batch2, gram_matrix, W1, b1,
           W2, b2, items):
    idx = rating_matrix_batch2.astype(jnp.int32)
    gathered = _make_sc_gather()(gram_matrix, idx)

    b1r = b1.reshape(1, 600)
    W2m = jnp.pad(W2[:, :LAT], ((0, 0), (0, LAT_PAD - LAT)))
    W2l = jnp.pad(W2[:, LAT:], ((0, 0), (0, LAT_PAD - LAT)))
    b2m = jnp.pad(b2[:LAT], (0, LAT_PAD - LAT)).reshape(1, LAT_PAD)
    b2l = jnp.pad(b2[LAT:], (0, LAT_PAD - LAT)).reshape(1, LAT_PAD)

    bi0, z_p, out, klp = _tc_encoder(
        gathered, rating_matrix_batch, W1, b1r, W2m, W2l, b2m, b2l, items)
    z = z_p[:, :LAT]
    kl = 0.5 * klp[0, 0] / BATCH_N
    return (z, out, kl, bi0)


# speedup vs baseline: 4.1172x; 1.3329x over previous
import functools

import jax
import jax.numpy as jnp
from jax import lax
from jax.experimental import pallas as pl
from jax.experimental.pallas import tpu as pltpu
from jax.experimental.pallas import tpu_sc as plsc

N_ITEMS = 8192
BATCH_N = 1024
H1_PAD = 640
LAT = 200
LAT_PAD = 256
TAU_C = 0.2

_NC = 2
_NS = 16
_NW = _NC * _NS
_BPW = BATCH_N // _NW
_CH = 8
_HALF = N_ITEMS // 2
_NCH = _BPW // _CH
_NU = _NCH * 2
_NBUF = 3


@functools.cache
def _make_sc_gather():
    @functools.partial(
        pl.kernel,
        mesh=plsc.VectorSubcoreMesh(core_axis_name="c", subcore_axis_name="s"),
        out_type=jax.ShapeDtypeStruct((BATCH_N, N_ITEMS), jnp.float32),
        scratch_types=[
            pltpu.VMEM((_BPW,), jnp.int32),
        ] + [pltpu.VMEM((_CH, _HALF), jnp.float32) for _ in range(_NBUF)]
          + [pltpu.SemaphoreType.DMA for _ in range(_NBUF)],
    )
    def _sc_gather(gram_hbm, idx_hbm, out_hbm, idx_v, *bufsems):
        bufs, sems = bufsems[:_NBUF], bufsems[_NBUF:]
        wid = lax.axis_index("s") * _NC + lax.axis_index("c")
        base = wid * _BPW

        def unit_src(u):
            c, h = u // 2, u % 2
            return gram_hbm.at[idx_v.at[pl.ds(c * _CH, _CH)],
                               pl.ds(h * _HALF, _HALF)]

        def unit_dst(u):
            c, h = u // 2, u % 2
            return out_hbm.at[pl.ds(base + c * _CH, _CH),
                              pl.ds(h * _HALF, _HALF)]

        pltpu.sync_copy(idx_hbm.at[pl.ds(base, _BPW)], idx_v)
        copies = [pltpu.async_copy(unit_src(u), bufs[u], sems[u])
                  for u in range(_NBUF)]
        for u in range(_NU):
            copies[u].wait()
            pltpu.sync_copy(bufs[u % _NBUF], unit_dst(u))
            if u + _NBUF < _NU:
                copies.append(pltpu.async_copy(
                    unit_src(u + _NBUF),
                    bufs[(u + _NBUF) % _NBUF], sems[(u + _NBUF) % _NBUF]))

    return _sc_gather


_MBLK = 128
_GRID = BATCH_N // _MBLK


def _dot_nt(a, b):
    return lax.dot_general(a, b, (((1,), (1,)), ((), ())),
                           preferred_element_type=jnp.float32)


def _dot_tn(a, b):
    return lax.dot_general(a, b, (((0,), (0,)), ((), ())),
                           preferred_element_type=jnp.float32)


def _tc_body(gath_ref, rate_ref, w1t_ref, b1c_ref, w2mt_ref, w2lt_ref,
             b2mc_ref, b2lc_ref, itT_ref, bi0_ref, zt_ref, out_ref, kl_ref,
             csc_ref):
    i = pl.program_id(0)

    @pl.when(i == 0)
    def _init():
        it = itT_ref[...]
        cn = jnp.sum(it * it, axis=0, keepdims=True)
        csc_ref[...] = 1.0 / jnp.maximum(jnp.sqrt(cn), 1e-12)
        kl_ref[...] = jnp.zeros((1, 1), jnp.float32)

    x = gath_ref[...] * (rate_ref[...] > 0).astype(jnp.float32)
    ss = jnp.sum(x * x, axis=1, keepdims=True)
    bi0 = x * (1.0 / jnp.maximum(jnp.sqrt(ss), 1e-12))
    bi0_ref[...] = bi0
    ht = jnp.tanh(_dot_nt(w1t_ref[...], bi0) + b1c_ref[...])
    meant = jnp.dot(w2mt_ref[...], ht,
                    preferred_element_type=jnp.float32) + b2mc_ref[...]
    logvart = jnp.dot(w2lt_ref[...], ht,
                      preferred_element_type=jnp.float32) + b2lc_ref[...]
    zt_ref[:, pl.ds(i * _MBLK, _MBLK)] = meant
    kl_ref[...] += jnp.sum(
        meant * meant + jnp.exp(logvart) - 1.0 - logvart).reshape(1, 1)
    zs = jnp.sum(meant * meant, axis=0, keepdims=True)
    znt = meant * (1.0 / jnp.maximum(jnp.sqrt(zs), 1e-12))
    out_ref[...] = (_dot_tn(znt, itT_ref[...])
                    * csc_ref[...]) * (1.0 / TAU_C)


def _tc_encoder(gathered, rating, W1T, b1c, W2mT, W2lT, b2mc, b2lc, itT):
    full = lambda shp: pl.BlockSpec(shp, lambda i: (0, 0))
    blk = lambda shp: pl.BlockSpec(shp, lambda i: (i, 0))
    return pl.pallas_call(
        _tc_body,
        grid=(_GRID,),
        in_specs=[
            blk((_MBLK, N_ITEMS)),
            blk((_MBLK, N_ITEMS)),
            full((600, N_ITEMS)),
            full((600, 1)),
            full((LAT, 600)),
            full((LAT, 600)),
            full((LAT, 1)),
            full((LAT, 1)),
            full((LAT, N_ITEMS)),
        ],
        out_specs=[
            blk((_MBLK, N_ITEMS)),
            pl.BlockSpec((LAT, BATCH_N), lambda i: (0, 0)),
            blk((_MBLK, N_ITEMS)),
            pl.BlockSpec((1, 1), lambda i: (0, 0)),
        ],
        out_shape=[
            jax.ShapeDtypeStruct((BATCH_N, N_ITEMS), jnp.float32),
            jax.ShapeDtypeStruct((LAT, BATCH_N), jnp.float32),
            jax.ShapeDtypeStruct((BATCH_N, N_ITEMS), jnp.float32),
            jax.ShapeDtypeStruct((1, 1), jnp.float32),
        ],
        scratch_shapes=[pltpu.VMEM((1, N_ITEMS), jnp.float32)],
        compiler_params=pltpu.CompilerParams(
            vmem_limit_bytes=100 * 1024 * 1024),
    )(gathered, rating, W1T, b1c, W2mT, W2lT, b2mc, b2lc, itT)


def kernel(rating_matrix_batch, rating_matrix_batch2, gram_matrix, W1, b1,
           W2, b2, items):
    idx = rating_matrix_batch2.astype(jnp.int32)
    gathered = _make_sc_gather()(gram_matrix, idx)

    W1T = W1.T
    itT = items.T
    b1c = b1.reshape(600, 1)
    W2T = W2.T
    W2mT = W2T[:LAT]
    W2lT = W2T[LAT:]
    b2mc = b2[:LAT].reshape(LAT, 1)
    b2lc = b2[LAT:].reshape(LAT, 1)

    bi0, zt_p, out, klp = _tc_encoder(
        gathered, rating_matrix_batch, W1T, b1c, W2mT, W2lT, b2mc, b2lc, itT)
    z = zt_p.T
    kl = 0.5 * klp[0, 0] / BATCH_N
    return (z, out, kl, bi0)
